# baseline (speedup 1.0000x reference)
import functools

import jax
import jax.numpy as jnp
from jax import lax
from jax.experimental import pallas as pl
from jax.experimental.pallas import tpu as pltpu
from jax.experimental.pallas import tpu_sc as plsc

NUM_USERS = 100000
NUM_BOOKS = 1000000
EMB = 64
BATCH = 16384

NC = 2
NS = 16
NW = NC * NS
ROWS_PER_W = BATCH // NW
CHUNK = 128
NCHUNK = ROWS_PER_W // CHUNK

BW = 65536


def _tc_scores(table_t, w_row, grid):
    def body(t_ref, w_ref, o_ref):
        o_ref[...] = jnp.dot(
            w_ref[...], t_ref[...],
            precision=lax.Precision.HIGHEST,
            preferred_element_type=jnp.float32,
        ).reshape(BW)

    return pl.pallas_call(
        body,
        grid=(grid,),
        in_specs=[
            pl.BlockSpec((EMB, BW), lambda t: (0, t)),
            pl.BlockSpec((1, EMB), lambda t: (0, 0)),
        ],
        out_specs=pl.BlockSpec((BW,), lambda t: (t,)),
        out_shape=jax.ShapeDtypeStruct((grid * BW,), jnp.float32),
    )(table_t, w_row)


def _sc_body(uid_hbm, bid_hbm, su_hbm, sb_hbm, bv_hbm, out_hbm,
             idx_u, idx_b, gu, gb, bv, out_v, sem):
    wid = lax.axis_index("s") * NC + lax.axis_index("c")
    base = wid * ROWS_PER_W

    pltpu.sync_copy(uid_hbm.at[wid], idx_u)
    pltpu.sync_copy(bid_hbm.at[wid], idx_b)
    pltpu.sync_copy(bv_hbm, bv)

    copies = []
    for j in range(NCHUNK):
        copies.append(pltpu.async_copy(su_hbm.at[idx_u.at[j]],
                                       gu.at[pl.ds(j * CHUNK, CHUNK)], sem))
        copies.append(pltpu.async_copy(sb_hbm.at[idx_b.at[j]],
                                       gb.at[pl.ds(j * CHUNK, CHUNK)], sem))
    for c in copies:
        c.wait()

    bias_v = bv[pl.ds(0, 16)]
    for k in range(ROWS_PER_W // 16):
        out_v[pl.ds(k * 16, 16)] = (gu[pl.ds(k * 16, 16)]
                                    + gb[pl.ds(k * 16, 16)] + bias_v)

    pltpu.sync_copy(out_v, out_hbm.at[pl.ds(base, ROWS_PER_W)])


@jax.jit
def kernel(user_ids, book_ids, user_table, book_table, W, b):
    uid3 = user_ids.reshape(NW, NCHUNK, CHUNK)
    bid3 = book_ids.reshape(NW, NCHUNK, CHUNK)
    wu = W[:, :EMB]
    wb = W[:, EMB:]
    b_vec = jnp.broadcast_to(b, (16,))

    gu = -(-NUM_USERS // BW)
    gb = -(-NUM_BOOKS // BW)
    scores_u = _tc_scores(user_table.T, wu, gu)
    scores_b = _tc_scores(book_table.T, wb, gb)

    mesh = plsc.VectorSubcoreMesh(core_axis_name="c", subcore_axis_name="s")
    fn = functools.partial(
        pl.kernel,
        mesh=mesh,
        out_type=jax.ShapeDtypeStruct((BATCH,), jnp.float32),
        scratch_types=[
            pltpu.VMEM((NCHUNK, CHUNK), jnp.int32),
            pltpu.VMEM((NCHUNK, CHUNK), jnp.int32),
            pltpu.VMEM((ROWS_PER_W,), jnp.float32),
            pltpu.VMEM((ROWS_PER_W,), jnp.float32),
            pltpu.VMEM((16,), jnp.float32),
            pltpu.VMEM((ROWS_PER_W,), jnp.float32),
            pltpu.SemaphoreType.DMA,
        ],
    )(_sc_body)
    return fn(uid3, bid3, scores_u, scores_b, b_vec)


# speedup vs baseline: 1.4097x; 1.4097x over previous
import functools

import jax
import jax.numpy as jnp
from jax import lax
from jax.experimental import pallas as pl
from jax.experimental.pallas import tpu as pltpu
from jax.experimental.pallas import tpu_sc as plsc

NUM_USERS = 100000
NUM_BOOKS = 1000000
EMB = 64
BATCH = 16384

NC = 2
NS = 16
NW = NC * NS
ROWS_PER_W = BATCH // NW
CHUNK = 128
NCHUNK = ROWS_PER_W // CHUNK

BW = 32768


def _tc_scores(table_t, w_row, grid):
    def body(t_ref, w_ref, o_ref):
        o_ref[...] = jnp.dot(
            w_ref[...], t_ref[...],
            preferred_element_type=jnp.float32,
        ).reshape(BW)

    return pl.pallas_call(
        body,
        grid=(grid,),
        in_specs=[
            pl.BlockSpec((EMB, BW), lambda t: (0, t)),
            pl.BlockSpec((1, EMB), lambda t: (0, 0)),
        ],
        out_specs=pl.BlockSpec((BW,), lambda t: (t,)),
        out_shape=jax.ShapeDtypeStruct((grid * BW,), jnp.float32),
    )(table_t, w_row)


def _sc_body(uid_hbm, bid_hbm, su_hbm, sb_hbm, bv_hbm, out_hbm,
             idx_u, idx_b, gu, gb, bv, out_v, sem):
    wid = lax.axis_index("s") * NC + lax.axis_index("c")
    base = wid * ROWS_PER_W

    pltpu.sync_copy(uid_hbm.at[wid], idx_u)
    pltpu.sync_copy(bid_hbm.at[wid], idx_b)
    pltpu.sync_copy(bv_hbm, bv)

    copies = []
    for j in range(NCHUNK):
        copies.append(pltpu.async_copy(su_hbm.at[idx_u.at[j]],
                                       gu.at[pl.ds(j * CHUNK, CHUNK)], sem))
        copies.append(pltpu.async_copy(sb_hbm.at[idx_b.at[j]],
                                       gb.at[pl.ds(j * CHUNK, CHUNK)], sem))
    for c in copies:
        c.wait()

    bias_v = bv[pl.ds(0, 16)]
    for k in range(ROWS_PER_W // 16):
        out_v[pl.ds(k * 16, 16)] = (gu[pl.ds(k * 16, 16)]
                                    + gb[pl.ds(k * 16, 16)] + bias_v)

    pltpu.sync_copy(out_v, out_hbm.at[pl.ds(base, ROWS_PER_W)])


@jax.jit
def kernel(user_ids, book_ids, user_table, book_table, W, b):
    uid3 = user_ids.reshape(NW, NCHUNK, CHUNK)
    bid3 = book_ids.reshape(NW, NCHUNK, CHUNK)
    wu = W[:, :EMB]
    wb = W[:, EMB:]
    b_vec = jnp.broadcast_to(b, (16,))

    gu = -(-NUM_USERS // BW)
    gb = -(-NUM_BOOKS // BW)
    scores_u = _tc_scores(user_table.T, wu, gu)
    scores_b = _tc_scores(book_table.T, wb, gb)

    mesh = plsc.VectorSubcoreMesh(core_axis_name="c", subcore_axis_name="s")
    fn = functools.partial(
        pl.kernel,
        mesh=mesh,
        out_type=jax.ShapeDtypeStruct((BATCH,), jnp.float32),
        scratch_types=[
            pltpu.VMEM((NCHUNK, CHUNK), jnp.int32),
            pltpu.VMEM((NCHUNK, CHUNK), jnp.int32),
            pltpu.VMEM((ROWS_PER_W,), jnp.float32),
            pltpu.VMEM((ROWS_PER_W,), jnp.float32),
            pltpu.VMEM((16,), jnp.float32),
            pltpu.VMEM((ROWS_PER_W,), jnp.float32),
            pltpu.SemaphoreType.DMA,
        ],
    )(_sc_body)
    return fn(uid3, bid3, scores_u, scores_b, b_vec)


# speedup vs baseline: 1.4220x; 1.0087x over previous
import functools

import jax
import jax.numpy as jnp
from jax import lax
from jax.experimental import pallas as pl
from jax.experimental.pallas import tpu as pltpu
from jax.experimental.pallas import tpu_sc as plsc

NUM_USERS = 100000
NUM_BOOKS = 1000000
EMB = 64
BATCH = 16384

NC = 2
NS = 16
NW = NC * NS
ROWS_PER_W = BATCH // NW
CHUNK = 128
NCHUNK = ROWS_PER_W // CHUNK

BW = 32768
GB = -(-NUM_BOOKS // BW)
GU = -(-NUM_USERS // BW)
USER_OFF = GB * BW


def _tc_scores(book_t, user_t, wb, wu):
    def body(bt_ref, ut_ref, wb_ref, wu_ref, o_ref):
        t = pl.program_id(0)

        @pl.when(t < GB)
        def _():
            o_ref[...] = jnp.dot(
                wb_ref[...], bt_ref[...],
                preferred_element_type=jnp.float32).reshape(BW)

        @pl.when(t >= GB)
        def _():
            o_ref[...] = jnp.dot(
                wu_ref[...], ut_ref[...],
                preferred_element_type=jnp.float32).reshape(BW)

    return pl.pallas_call(
        body,
        grid=(GB + GU,),
        in_specs=[
            pl.BlockSpec((EMB, BW), lambda t: (0, jnp.minimum(t, GB - 1))),
            pl.BlockSpec((EMB, BW),
                         lambda t: (0, jnp.clip(t - GB, 0, GU - 1))),
            pl.BlockSpec((1, EMB), lambda t: (0, 0)),
            pl.BlockSpec((1, EMB), lambda t: (0, 0)),
        ],
        out_specs=pl.BlockSpec((BW,), lambda t: (t,)),
        out_shape=jax.ShapeDtypeStruct(((GB + GU) * BW,), jnp.float32),
    )(book_t, user_t, wb, wu)


def _sc_body(uid_hbm, bid_hbm, sc_hbm, bv_hbm, out_hbm,
             idx_u, idx_b, gu, gb, bv, out_v, sem):
    wid = lax.axis_index("s") * NC + lax.axis_index("c")
    base = wid * ROWS_PER_W

    pltpu.sync_copy(uid_hbm.at[wid], idx_u)
    pltpu.sync_copy(bid_hbm.at[wid], idx_b)
    pltpu.sync_copy(bv_hbm, bv)

    for j in range(NCHUNK):
        for k in range(CHUNK // 16):
            idx_u[j, pl.ds(k * 16, 16)] = idx_u[j, pl.ds(k * 16, 16)] + USER_OFF

    copies = []
    for j in range(NCHUNK):
        copies.append(pltpu.async_copy(sc_hbm.at[idx_u.at[j]],
                                       gu.at[pl.ds(j * CHUNK, CHUNK)], sem))
        copies.append(pltpu.async_copy(sc_hbm.at[idx_b.at[j]],
                                       gb.at[pl.ds(j * CHUNK, CHUNK)], sem))
    for c in copies:
        c.wait()

    bias_v = bv[pl.ds(0, 16)]
    for k in range(ROWS_PER_W // 16):
        out_v[pl.ds(k * 16, 16)] = (gu[pl.ds(k * 16, 16)]
                                    + gb[pl.ds(k * 16, 16)] + bias_v)

    pltpu.sync_copy(out_v, out_hbm.at[pl.ds(base, ROWS_PER_W)])


@jax.jit
def kernel(user_ids, book_ids, user_table, book_table, W, b):
    uid3 = user_ids.reshape(NW, NCHUNK, CHUNK)
    bid3 = book_ids.reshape(NW, NCHUNK, CHUNK)
    wu = W[:, :EMB]
    wb = W[:, EMB:]
    b_vec = jnp.broadcast_to(b, (16,))

    scores = _tc_scores(book_table.T, user_table.T, wb, wu)

    mesh = plsc.VectorSubcoreMesh(core_axis_name="c", subcore_axis_name="s")
    fn = functools.partial(
        pl.kernel,
        mesh=mesh,
        out_type=jax.ShapeDtypeStruct((BATCH,), jnp.float32),
        scratch_types=[
            pltpu.VMEM((NCHUNK, CHUNK), jnp.int32),
            pltpu.VMEM((NCHUNK, CHUNK), jnp.int32),
            pltpu.VMEM((ROWS_PER_W,), jnp.float32),
            pltpu.VMEM((ROWS_PER_W,), jnp.float32),
            pltpu.VMEM((16,), jnp.float32),
            pltpu.VMEM((ROWS_PER_W,), jnp.float32),
            pltpu.SemaphoreType.DMA,
        ],
    )(_sc_body)
    return fn(uid3, bid3, scores, b_vec)
